# baseline (speedup 1.0000x reference)
import functools

import jax
import jax.numpy as jnp
from jax import lax
from jax.experimental import pallas as pl
from jax.experimental.pallas import tpu as pltpu
from jax.experimental.pallas import tpu_sc as plsc

T = 2048
H = 1024
E = 16
N_GROUP = 4
GSZ = E // N_GROUP
I = 512
IS = 1024
ROUTED_SCALING = 2.5

BM = 256
NTILES = 32
PAD_T = NTILES * BM

NC = 2
NS = 16
NW = NC * NS
TPW = T // NW
CH = 16

_NEG = -1e30


def _routing_body(x_ref, gw_ref, gb_ref,
                  p1_ref, p2_ref, w1_ref, w2_ref, te_ref, used_ref):
    x = x_ref[...]
    gw = gw_ref[...]
    gb = gb_ref[...]
    logits = lax.dot_general(x, gw, (((1,), (1,)), ((), ())),
                             preferred_element_type=jnp.float32)
    logits = logits + gb
    scores = jax.nn.sigmoid(logits)
    sfc = scores + gb

    il = lax.broadcasted_iota(jnp.int32, (T, E), 1)
    gid = il // GSZ

    gs_cols = []
    for g in range(N_GROUP):
        mg = gid == g
        m1 = jnp.max(jnp.where(mg, sfc, _NEG), axis=1, keepdims=True)
        idx1 = jnp.min(jnp.where(mg & (sfc == m1), il, 99), axis=1,
                       keepdims=True)
        m2 = jnp.max(jnp.where(mg & (il != idx1), sfc, _NEG), axis=1,
                     keepdims=True)
        gs_cols.append(m1 + m2)
    gs = jnp.concatenate(gs_cols, axis=1)

    i4 = lax.broadcasted_iota(jnp.int32, (T, N_GROUP), 1)
    gm1 = jnp.max(gs, axis=1, keepdims=True)
    gi1 = jnp.min(jnp.where(gs == gm1, i4, 99), axis=1, keepdims=True)
    gs2 = jnp.where(i4 == gi1, _NEG, gs)
    gm2 = jnp.max(gs2, axis=1, keepdims=True)
    gi2 = jnp.min(jnp.where(gs2 == gm2, i4, 99), axis=1, keepdims=True)

    allowed = (gid == gi1) | (gid == gi2)
    msk = jnp.where(allowed, sfc, _NEG)

    v1 = jnp.max(msk, axis=1, keepdims=True)
    e1 = jnp.min(jnp.where(msk == v1, il, 99), axis=1, keepdims=True)
    msk2 = jnp.where(il == e1, _NEG, msk)
    v2 = jnp.max(msk2, axis=1, keepdims=True)
    e2 = jnp.min(jnp.where(msk2 == v2, il, 99), axis=1, keepdims=True)

    w1 = jnp.sum(jnp.where(il == e1, scores, 0.0), axis=1, keepdims=True)
    w2 = jnp.sum(jnp.where(il == e2, scores, 0.0), axis=1, keepdims=True)
    denom = w1 + w2 + 1e-20
    w1_ref[...] = ROUTED_SCALING * w1 / denom
    w2_ref[...] = ROUTED_SCALING * w2 / denom

    sel1 = jnp.where(il == e1, 1.0, 0.0)
    sel2 = jnp.where(il == e2, 1.0, 0.0)
    s = sel1 + sel2
    excl = jnp.concatenate([jnp.zeros((1, E), jnp.float32), s[:-1]], axis=0)
    k = 1
    while k < T:
        shifted = jnp.concatenate(
            [jnp.zeros((k, E), jnp.float32), excl[:T - k]], axis=0)
        excl = excl + shifted
        k *= 2
    counts = jnp.sum(s, axis=0, keepdims=True)
    tiles = (counts.astype(jnp.int32) + (BM - 1)) // BM
    tiles_f = tiles.astype(jnp.float32)
    r16 = lax.broadcasted_iota(jnp.int32, (E, E), 0)
    c16 = lax.broadcasted_iota(jnp.int32, (E, E), 1)
    mlt = jnp.where(r16 < c16, 1.0, 0.0)
    base = lax.dot_general(tiles_f, mlt, (((1,), (0,)), ((), ())),
                           preferred_element_type=jnp.float32)
    aligned = base * float(BM)

    rank1 = jnp.sum(jnp.where(il == e1, excl, 0.0), axis=1, keepdims=True)
    off1 = jnp.sum(jnp.where(il == e1, aligned, 0.0), axis=1, keepdims=True)
    p1_ref[...] = (rank1 + off1).astype(jnp.int32)
    rank2 = jnp.sum(jnp.where(il == e2, excl, 0.0), axis=1, keepdims=True)
    off2 = jnp.sum(jnp.where(il == e2, aligned, 0.0), axis=1, keepdims=True)
    p2_ref[...] = (rank2 + off2).astype(jnp.int32)

    used_ref[...] = jnp.sum(tiles, axis=1, keepdims=True)

    it = lax.broadcasted_iota(jnp.int32, (NTILES, 1), 0).astype(jnp.float32)
    cnt = jnp.sum(jnp.where(base <= it, 1, 0), axis=1, keepdims=True)
    te_ref[...] = cnt - 1


def _dispatch_body(p1_hbm, p2_hbm, w1_hbm, w2_hbm,
                   gidx_hbm, wgt_hbm, idx1_v, idx2_v, wv1_v, wv2_v, tid_v,
                   sem1, sem2, sem3, sem4):
    wid = lax.axis_index("s") * NC + lax.axis_index("c")
    base = wid * TPW
    sl = pl.ds(base, TPW)
    pltpu.sync_copy(p1_hbm.at[sl], idx1_v)
    pltpu.sync_copy(p2_hbm.at[sl], idx2_v)
    pltpu.sync_copy(w1_hbm.at[sl], wv1_v)
    pltpu.sync_copy(w2_hbm.at[sl], wv2_v)
    for k in range(TPW // 16):
        tid_v[pl.ds(k * 16, 16)] = (
            lax.broadcasted_iota(jnp.int32, (16,), 0) + (base + k * 16))
    c1 = pltpu.async_copy(tid_v, gidx_hbm.at[idx1_v], sem1)
    c2 = pltpu.async_copy(tid_v, gidx_hbm.at[idx2_v], sem2)
    c3 = pltpu.async_copy(wv1_v, wgt_hbm.at[idx1_v], sem3)
    c4 = pltpu.async_copy(wv2_v, wgt_hbm.at[idx2_v], sem4)
    c1.wait()
    c2.wait()
    c3.wait()
    c4.wait()


def _gather_rows_body(xi_hbm, gidx_hbm, used_hbm, xg_hbm,
                      gi_v, rows_v, used_v, sem):
    wid = lax.axis_index("s") * NC + lax.axis_index("c")
    pltpu.sync_copy(used_hbm, used_v)
    used_s = used_v[pl.ds(0, 16)][0]

    @pl.when(wid < used_s)
    def _():
        pltpu.sync_copy(gidx_hbm.at[pl.ds(wid * BM, BM)], gi_v)
        for k in range(BM // 16):
            slk = pl.ds(k * 16, 16)
            gi_v[slk] = jnp.minimum(jnp.maximum(gi_v[slk], 0), T - 1)
        for c in range(2):
            pltpu.async_copy(
                xi_hbm.at[gi_v.at[pl.ds(c * (BM // 2), BM // 2)]],
                rows_v, sem).wait()
            pltpu.sync_copy(
                rows_v, xg_hbm.at[pl.ds(wid * BM + c * (BM // 2), BM // 2)])


def _gmm_body(te_ref, used_ref, xg_ref, wgu_ref, wd_ref, wgt_ref, yg_ref):
    i = pl.program_id(0)

    @pl.when(i < used_ref[0])
    def _():
        xi = xg_ref[...]
        xlo = lax.bitcast_convert_type(
            lax.shift_left(xi, 16), jnp.float32).astype(jnp.bfloat16)
        xhi = lax.bitcast_convert_type(
            xi & jnp.int32(-65536), jnp.float32).astype(jnp.bfloat16)
        wgu = wgu_ref[0]
        gu = (lax.dot_general(xlo, wgu[:, :H // 2],
                              (((1,), (1,)), ((), ())),
                              preferred_element_type=jnp.float32)
              + lax.dot_general(xhi, wgu[:, H // 2:],
                                (((1,), (1,)), ((), ())),
                                preferred_element_type=jnp.float32))
        g = gu[:, :I]
        u = gu[:, I:]
        h = (g * jax.nn.sigmoid(g) * u).astype(jnp.bfloat16)
        wd = wd_ref[0]
        eo = lax.dot_general(h, wd, (((1,), (1,)), ((), ())),
                             preferred_element_type=jnp.float32)
        yg_ref[...] = eo * wgt_ref[...]


def _shared_body(x_ref, sgu_ref, sd_ref, out_ref):
    x = x_ref[...]
    sgu = sgu_ref[...]
    gu = lax.dot_general(x, sgu, (((1,), (1,)), ((), ())),
                         preferred_element_type=jnp.float32)
    g = gu[:, :IS]
    u = gu[:, IS:]
    h = (g * jax.nn.sigmoid(g) * u).astype(jnp.bfloat16)
    sd = sd_ref[...]
    out_ref[...] = lax.dot_general(h, sd, (((1,), (1,)), ((), ())),
                                   preferred_element_type=jnp.float32)


def _combine_body(yg_hbm, p1_hbm, p2_hbm, sh_hbm, out_hbm,
                  idx1_v, idx2_v, y1_v, y2_v, sh_v, ob_v, sem, sem2, sem3):
    wid = lax.axis_index("s") * NC + lax.axis_index("c")
    base = wid * TPW
    for c in range(TPW // CH):
        b = base + c * CH
        pltpu.sync_copy(p1_hbm.at[pl.ds(b, CH)], idx1_v)
        pltpu.sync_copy(p2_hbm.at[pl.ds(b, CH)], idx2_v)
        c1 = pltpu.async_copy(yg_hbm.at[idx1_v], y1_v, sem)
        c2 = pltpu.async_copy(yg_hbm.at[idx2_v], y2_v, sem2)
        c3 = pltpu.async_copy(sh_hbm.at[pl.ds(b, CH)], sh_v, sem3)
        c1.wait()
        c2.wait()
        c3.wait()

        def body_j(j, carry):
            def body_v(v, carry2):
                sl = pl.ds(v * 16, 16)
                ob_v[j, sl] = y1_v[j, sl] + y2_v[j, sl] + sh_v[j, sl]
                return carry2
            return lax.fori_loop(0, H // 16, body_v, carry)
        lax.fori_loop(0, CH, body_j, 0)
        pltpu.sync_copy(ob_v, out_hbm.at[pl.ds(b, CH)])


@jax.jit
def kernel(hidden_states, gate_w, gate_b, w_gate_up, w_down,
           shared_gate_up, shared_down):
    x = hidden_states
    xb = x.astype(jnp.bfloat16)
    wgu = w_gate_up.astype(jnp.bfloat16)
    wd = w_down.astype(jnp.bfloat16)
    sgu = shared_gate_up.astype(jnp.bfloat16)
    sd = shared_down.astype(jnp.bfloat16)

    p1, p2, w1, w2, te, used = pl.pallas_call(
        _routing_body,
        out_shape=(
            jax.ShapeDtypeStruct((T, 1), jnp.int32),
            jax.ShapeDtypeStruct((T, 1), jnp.int32),
            jax.ShapeDtypeStruct((T, 1), jnp.float32),
            jax.ShapeDtypeStruct((T, 1), jnp.float32),
            jax.ShapeDtypeStruct((NTILES, 1), jnp.int32),
            jax.ShapeDtypeStruct((1, 1), jnp.int32),
        ),
    )(x, gate_w, gate_b.reshape(1, E))

    p1f = p1.reshape(T)
    p2f = p2.reshape(T)

    mesh = plsc.VectorSubcoreMesh(core_axis_name="c", subcore_axis_name="s")
    gidx, wgt = pl.kernel(
        _dispatch_body,
        mesh=mesh,
        out_type=(
            jax.ShapeDtypeStruct((PAD_T,), jnp.int32),
            jax.ShapeDtypeStruct((PAD_T,), jnp.float32),
        ),
        scratch_types=[
            pltpu.VMEM((TPW,), jnp.int32),
            pltpu.VMEM((TPW,), jnp.int32),
            pltpu.VMEM((TPW,), jnp.float32),
            pltpu.VMEM((TPW,), jnp.float32),
            pltpu.VMEM((TPW,), jnp.int32),
            pltpu.SemaphoreType.DMA,
            pltpu.SemaphoreType.DMA,
            pltpu.SemaphoreType.DMA,
            pltpu.SemaphoreType.DMA,
        ],
    )(p1f, p2f, w1.reshape(T), w2.reshape(T))

    xi = lax.bitcast_convert_type(
        jnp.stack([xb[:, :H // 2], xb[:, H // 2:]], axis=-1), jnp.int32)
    xg = pl.kernel(
        _gather_rows_body,
        mesh=mesh,
        out_type=jax.ShapeDtypeStruct((PAD_T, H // 2), jnp.int32),
        scratch_types=[
            pltpu.VMEM((BM,), jnp.int32),
            pltpu.VMEM((BM // 2, H // 2), jnp.int32),
            pltpu.VMEM((16,), jnp.int32),
            pltpu.SemaphoreType.DMA,
        ],
    )(xi, gidx, jnp.broadcast_to(used.reshape(1), (16,)))

    yg = pl.pallas_call(
        _gmm_body,
        grid_spec=pltpu.PrefetchScalarGridSpec(
            num_scalar_prefetch=2,
            grid=(NTILES,),
            in_specs=[
                pl.BlockSpec((BM, H // 2), lambda i, te, u: (i, 0)),
                pl.BlockSpec((1, 2 * I, H), lambda i, te, u: (te[i], 0, 0)),
                pl.BlockSpec((1, H, I), lambda i, te, u: (te[i], 0, 0)),
                pl.BlockSpec((BM, 1), lambda i, te, u: (i, 0)),
            ],
            out_specs=pl.BlockSpec((BM, H), lambda i, te, u: (i, 0)),
        ),
        out_shape=jax.ShapeDtypeStruct((PAD_T, H), jnp.float32),
        compiler_params=pltpu.CompilerParams(
            dimension_semantics=("arbitrary",)),
    )(te.reshape(NTILES), used.reshape(1), xg, wgu, wd,
      wgt.reshape(PAD_T, 1))

    shared_out = pl.pallas_call(
        _shared_body,
        grid=(T // 256,),
        in_specs=[
            pl.BlockSpec((256, H), lambda t: (t, 0)),
            pl.BlockSpec((2 * IS, H), lambda t: (0, 0)),
            pl.BlockSpec((H, IS), lambda t: (0, 0)),
        ],
        out_specs=pl.BlockSpec((256, H), lambda t: (t, 0)),
        out_shape=jax.ShapeDtypeStruct((T, H), jnp.float32),
        compiler_params=pltpu.CompilerParams(
            dimension_semantics=("arbitrary",)),
    )(xb, sgu, sd)

    out = pl.kernel(
        _combine_body,
        mesh=mesh,
        out_type=jax.ShapeDtypeStruct((T, H), jnp.float32),
        scratch_types=[
            pltpu.VMEM((CH,), jnp.int32),
            pltpu.VMEM((CH,), jnp.int32),
            pltpu.VMEM((CH, H), jnp.float32),
            pltpu.VMEM((CH, H), jnp.float32),
            pltpu.VMEM((CH, H), jnp.float32),
            pltpu.VMEM((CH, H), jnp.float32),
            pltpu.SemaphoreType.DMA,
            pltpu.SemaphoreType.DMA,
            pltpu.SemaphoreType.DMA,
        ],
    )(yg, p1f, p2f, shared_out)

    return out


# speedup vs baseline: 1.3624x; 1.3624x over previous
import functools

import jax
import jax.numpy as jnp
from jax import lax
from jax.experimental import pallas as pl
from jax.experimental.pallas import tpu as pltpu
from jax.experimental.pallas import tpu_sc as plsc

T = 2048
H = 1024
E = 16
N_GROUP = 4
GSZ = E // N_GROUP
I = 512
IS = 1024
ROUTED_SCALING = 2.5

BM = 256
NTILES = 32
PAD_T = NTILES * BM

NC = 2
NS = 16
NW = NC * NS
TPW = T // NW
CH = 16

_NEG = -1e30


def _routing_body(x_ref, gw_ref, gb_ref,
                  p1_ref, p2_ref, w1_ref, w2_ref, te_ref, used_ref):
    x = x_ref[...]
    gw = gw_ref[...]
    gb = gb_ref[...]
    logits = lax.dot_general(x, gw, (((1,), (1,)), ((), ())),
                             preferred_element_type=jnp.float32)
    logits = logits + gb
    scores = jax.nn.sigmoid(logits)
    sfc = scores + gb

    il = lax.broadcasted_iota(jnp.int32, (T, E), 1)
    gid = il // GSZ

    gs_cols = []
    for g in range(N_GROUP):
        mg = gid == g
        m1 = jnp.max(jnp.where(mg, sfc, _NEG), axis=1, keepdims=True)
        idx1 = jnp.min(jnp.where(mg & (sfc == m1), il, 99), axis=1,
                       keepdims=True)
        m2 = jnp.max(jnp.where(mg & (il != idx1), sfc, _NEG), axis=1,
                     keepdims=True)
        gs_cols.append(m1 + m2)
    gs = jnp.concatenate(gs_cols, axis=1)

    i4 = lax.broadcasted_iota(jnp.int32, (T, N_GROUP), 1)
    gm1 = jnp.max(gs, axis=1, keepdims=True)
    gi1 = jnp.min(jnp.where(gs == gm1, i4, 99), axis=1, keepdims=True)
    gs2 = jnp.where(i4 == gi1, _NEG, gs)
    gm2 = jnp.max(gs2, axis=1, keepdims=True)
    gi2 = jnp.min(jnp.where(gs2 == gm2, i4, 99), axis=1, keepdims=True)

    allowed = (gid == gi1) | (gid == gi2)
    msk = jnp.where(allowed, sfc, _NEG)

    v1 = jnp.max(msk, axis=1, keepdims=True)
    e1 = jnp.min(jnp.where(msk == v1, il, 99), axis=1, keepdims=True)
    msk2 = jnp.where(il == e1, _NEG, msk)
    v2 = jnp.max(msk2, axis=1, keepdims=True)
    e2 = jnp.min(jnp.where(msk2 == v2, il, 99), axis=1, keepdims=True)

    w1 = jnp.sum(jnp.where(il == e1, scores, 0.0), axis=1, keepdims=True)
    w2 = jnp.sum(jnp.where(il == e2, scores, 0.0), axis=1, keepdims=True)
    denom = w1 + w2 + 1e-20
    w1_ref[...] = ROUTED_SCALING * w1 / denom
    w2_ref[...] = ROUTED_SCALING * w2 / denom

    sel1 = jnp.where(il == e1, 1.0, 0.0)
    sel2 = jnp.where(il == e2, 1.0, 0.0)
    s = sel1 + sel2
    excl = jnp.concatenate([jnp.zeros((1, E), jnp.float32), s[:-1]], axis=0)
    k = 1
    while k < T:
        shifted = jnp.concatenate(
            [jnp.zeros((k, E), jnp.float32), excl[:T - k]], axis=0)
        excl = excl + shifted
        k *= 2
    counts = jnp.sum(s, axis=0, keepdims=True)
    tiles = (counts.astype(jnp.int32) + (BM - 1)) // BM
    tiles_f = tiles.astype(jnp.float32)
    r16 = lax.broadcasted_iota(jnp.int32, (E, E), 0)
    c16 = lax.broadcasted_iota(jnp.int32, (E, E), 1)
    mlt = jnp.where(r16 < c16, 1.0, 0.0)
    base = lax.dot_general(tiles_f, mlt, (((1,), (0,)), ((), ())),
                           preferred_element_type=jnp.float32)
    aligned = base * float(BM)

    rank1 = jnp.sum(jnp.where(il == e1, excl, 0.0), axis=1, keepdims=True)
    off1 = jnp.sum(jnp.where(il == e1, aligned, 0.0), axis=1, keepdims=True)
    p1_ref[...] = (rank1 + off1).astype(jnp.int32)
    rank2 = jnp.sum(jnp.where(il == e2, excl, 0.0), axis=1, keepdims=True)
    off2 = jnp.sum(jnp.where(il == e2, aligned, 0.0), axis=1, keepdims=True)
    p2_ref[...] = (rank2 + off2).astype(jnp.int32)

    used_ref[...] = jnp.sum(tiles, axis=1, keepdims=True)

    it = lax.broadcasted_iota(jnp.int32, (NTILES, 1), 0).astype(jnp.float32)
    cnt = jnp.sum(jnp.where(base <= it, 1, 0), axis=1, keepdims=True)
    te_ref[...] = cnt - 1


def _scatter_body(x_hbm, p1_hbm, p2_hbm, xg_hbm,
                  idx1_v, idx2_v, rows_v, sem1, sem2):
    wid = lax.axis_index("s") * NC + lax.axis_index("c")
    base = wid * TPW
    sl = pl.ds(base, TPW)
    pltpu.sync_copy(p1_hbm.at[sl], idx1_v)
    pltpu.sync_copy(p2_hbm.at[sl], idx2_v)
    pltpu.sync_copy(x_hbm.at[sl], rows_v)
    c1 = pltpu.async_copy(rows_v, xg_hbm.at[idx1_v], sem1)
    c2 = pltpu.async_copy(rows_v, xg_hbm.at[idx2_v], sem2)
    c1.wait()
    c2.wait()


def _gmm_body(te_ref, used_ref, xg_ref, wgu_ref, wd_ref, yg_ref):
    i = pl.program_id(0)

    @pl.when(i < used_ref[0])
    def _():
        xi = xg_ref[...]
        xlo = lax.bitcast_convert_type(
            lax.shift_left(xi, 16), jnp.float32).astype(jnp.bfloat16)
        xhi = lax.bitcast_convert_type(
            xi & jnp.int32(-65536), jnp.float32).astype(jnp.bfloat16)
        wgu = wgu_ref[0]
        gu = (lax.dot_general(xlo, wgu[:, :H // 2],
                              (((1,), (1,)), ((), ())),
                              preferred_element_type=jnp.float32)
              + lax.dot_general(xhi, wgu[:, H // 2:],
                                (((1,), (1,)), ((), ())),
                                preferred_element_type=jnp.float32))
        g = gu[:, :I]
        u = gu[:, I:]
        h = (g * jax.nn.sigmoid(g) * u).astype(jnp.bfloat16)
        wd = wd_ref[0]
        eo = lax.dot_general(h, wd, (((1,), (1,)), ((), ())),
                             preferred_element_type=jnp.float32)
        yg_ref[...] = eo


def _shared_body(x_ref, sgu_ref, sd_ref, out_ref):
    x = x_ref[...]
    sgu = sgu_ref[...]
    gu = lax.dot_general(x, sgu, (((1,), (1,)), ((), ())),
                         preferred_element_type=jnp.float32)
    g = gu[:, :IS]
    u = gu[:, IS:]
    h = (g * jax.nn.sigmoid(g) * u).astype(jnp.bfloat16)
    sd = sd_ref[...]
    out_ref[...] = lax.dot_general(h, sd, (((1,), (1,)), ((), ())),
                                   preferred_element_type=jnp.float32)


def _combine_body(yg_hbm, p1_hbm, p2_hbm, w1_hbm, w2_hbm, sh_hbm, out_hbm,
                  idx1_v, idx2_v, wv1_v, wv2_v, y1_v, y2_v, sh_v, ob_v,
                  sem, sem2, sem3):
    wid = lax.axis_index("s") * NC + lax.axis_index("c")
    base = wid * TPW
    for c in range(TPW // CH):
        b = base + c * CH
        pltpu.sync_copy(p1_hbm.at[pl.ds(b, CH)], idx1_v)
        pltpu.sync_copy(p2_hbm.at[pl.ds(b, CH)], idx2_v)
        pltpu.sync_copy(w1_hbm.at[pl.ds(b, CH)], wv1_v)
        pltpu.sync_copy(w2_hbm.at[pl.ds(b, CH)], wv2_v)
        c1 = pltpu.async_copy(yg_hbm.at[idx1_v], y1_v, sem)
        c2 = pltpu.async_copy(yg_hbm.at[idx2_v], y2_v, sem2)
        c3 = pltpu.async_copy(sh_hbm.at[pl.ds(b, CH)], sh_v, sem3)
        c1.wait()
        c2.wait()
        c3.wait()

        wa = wv1_v[pl.ds(0, CH)]
        wb = wv2_v[pl.ds(0, CH)]

        def body_v(v, carry):
            sl = pl.ds(v * 16, 16)
            for j in range(CH):
                ob_v[j, sl] = (y1_v[j, sl] * wa[j] + y2_v[j, sl] * wb[j]
                               + sh_v[j, sl])
            return carry
        lax.fori_loop(0, H // 16, body_v, 0)
        pltpu.sync_copy(ob_v, out_hbm.at[pl.ds(b, CH)])


@jax.jit
def kernel(hidden_states, gate_w, gate_b, w_gate_up, w_down,
           shared_gate_up, shared_down):
    x = hidden_states
    xb = x.astype(jnp.bfloat16)
    wgu = w_gate_up.astype(jnp.bfloat16)
    wd = w_down.astype(jnp.bfloat16)
    sgu = shared_gate_up.astype(jnp.bfloat16)
    sd = shared_down.astype(jnp.bfloat16)

    p1, p2, w1, w2, te, used = pl.pallas_call(
        _routing_body,
        out_shape=(
            jax.ShapeDtypeStruct((T, 1), jnp.int32),
            jax.ShapeDtypeStruct((T, 1), jnp.int32),
            jax.ShapeDtypeStruct((T, 1), jnp.float32),
            jax.ShapeDtypeStruct((T, 1), jnp.float32),
            jax.ShapeDtypeStruct((NTILES, 1), jnp.int32),
            jax.ShapeDtypeStruct((1, 1), jnp.int32),
        ),
    )(x, gate_w, gate_b.reshape(1, E))

    p1f = p1.reshape(T)
    p2f = p2.reshape(T)

    mesh = plsc.VectorSubcoreMesh(core_axis_name="c", subcore_axis_name="s")
    xi = lax.bitcast_convert_type(
        jnp.stack([xb[:, :H // 2], xb[:, H // 2:]], axis=-1), jnp.int32)
    xg = pl.kernel(
        _scatter_body,
        mesh=mesh,
        out_type=jax.ShapeDtypeStruct((PAD_T, H // 2), jnp.int32),
        scratch_types=[
            pltpu.VMEM((TPW,), jnp.int32),
            pltpu.VMEM((TPW,), jnp.int32),
            pltpu.VMEM((TPW, H // 2), jnp.int32),
            pltpu.SemaphoreType.DMA,
            pltpu.SemaphoreType.DMA,
        ],
    )(xi, p1f, p2f)

    yg = pl.pallas_call(
        _gmm_body,
        grid_spec=pltpu.PrefetchScalarGridSpec(
            num_scalar_prefetch=2,
            grid=(NTILES,),
            in_specs=[
                pl.BlockSpec((BM, H // 2), lambda i, te, u: (i, 0)),
                pl.BlockSpec((1, 2 * I, H), lambda i, te, u: (te[i], 0, 0)),
                pl.BlockSpec((1, H, I), lambda i, te, u: (te[i], 0, 0)),
            ],
            out_specs=pl.BlockSpec((BM, H), lambda i, te, u: (i, 0)),
        ),
        out_shape=jax.ShapeDtypeStruct((PAD_T, H), jnp.float32),
        compiler_params=pltpu.CompilerParams(
            dimension_semantics=("arbitrary",)),
    )(te.reshape(NTILES), used.reshape(1), xg, wgu, wd)

    shared_out = pl.pallas_call(
        _shared_body,
        grid=(T // 256,),
        in_specs=[
            pl.BlockSpec((256, H), lambda t: (t, 0)),
            pl.BlockSpec((2 * IS, H), lambda t: (0, 0)),
            pl.BlockSpec((H, IS), lambda t: (0, 0)),
        ],
        out_specs=pl.BlockSpec((256, H), lambda t: (t, 0)),
        out_shape=jax.ShapeDtypeStruct((T, H), jnp.float32),
        compiler_params=pltpu.CompilerParams(
            dimension_semantics=("arbitrary",)),
    )(xb, sgu, sd)

    out = pl.kernel(
        _combine_body,
        mesh=mesh,
        out_type=jax.ShapeDtypeStruct((T, H), jnp.float32),
        scratch_types=[
            pltpu.VMEM((CH,), jnp.int32),
            pltpu.VMEM((CH,), jnp.int32),
            pltpu.VMEM((CH,), jnp.float32),
            pltpu.VMEM((CH,), jnp.float32),
            pltpu.VMEM((CH, H), jnp.float32),
            pltpu.VMEM((CH, H), jnp.float32),
            pltpu.VMEM((CH, H), jnp.float32),
            pltpu.VMEM((CH, H), jnp.float32),
            pltpu.SemaphoreType.DMA,
            pltpu.SemaphoreType.DMA,
            pltpu.SemaphoreType.DMA,
        ],
    )(yg, p1f, p2f, w1.reshape(T), w2.reshape(T), shared_out)

    return out


# speedup vs baseline: 1.3959x; 1.0246x over previous
import functools

import jax
import jax.numpy as jnp
from jax import lax
from jax.experimental import pallas as pl
from jax.experimental.pallas import tpu as pltpu
from jax.experimental.pallas import tpu_sc as plsc

T = 2048
H = 1024
E = 16
N_GROUP = 4
GSZ = E // N_GROUP
I = 512
IS = 1024
ROUTED_SCALING = 2.5

BM = 256
NTILES = 32
PAD_T = NTILES * BM

NC = 2
NS = 16
NW = NC * NS
TPW = T // NW
CH = 16

_NEG = -1e30


def _routing_body(x_ref, gw_ref, gb_ref,
                  p1_ref, p2_ref, w1_ref, w2_ref, te_ref, used_ref, xi_ref):
    x = x_ref[...]
    gw = gw_ref[...]
    gb = gb_ref[...]
    logits = lax.dot_general(x, gw, (((1,), (1,)), ((), ())),
                             preferred_element_type=jnp.float32)
    logits = logits + gb
    scores = jax.nn.sigmoid(logits)
    sfc = scores + gb

    il = lax.broadcasted_iota(jnp.int32, (T, E), 1)
    gid = il // GSZ

    gs_cols = []
    for g in range(N_GROUP):
        mg = gid == g
        m1 = jnp.max(jnp.where(mg, sfc, _NEG), axis=1, keepdims=True)
        idx1 = jnp.min(jnp.where(mg & (sfc == m1), il, 99), axis=1,
                       keepdims=True)
        m2 = jnp.max(jnp.where(mg & (il != idx1), sfc, _NEG), axis=1,
                     keepdims=True)
        gs_cols.append(m1 + m2)
    gs = jnp.concatenate(gs_cols, axis=1)

    i4 = lax.broadcasted_iota(jnp.int32, (T, N_GROUP), 1)
    gm1 = jnp.max(gs, axis=1, keepdims=True)
    gi1 = jnp.min(jnp.where(gs == gm1, i4, 99), axis=1, keepdims=True)
    gs2 = jnp.where(i4 == gi1, _NEG, gs)
    gm2 = jnp.max(gs2, axis=1, keepdims=True)
    gi2 = jnp.min(jnp.where(gs2 == gm2, i4, 99), axis=1, keepdims=True)

    allowed = (gid == gi1) | (gid == gi2)
    msk = jnp.where(allowed, sfc, _NEG)

    v1 = jnp.max(msk, axis=1, keepdims=True)
    e1 = jnp.min(jnp.where(msk == v1, il, 99), axis=1, keepdims=True)
    msk2 = jnp.where(il == e1, _NEG, msk)
    v2 = jnp.max(msk2, axis=1, keepdims=True)
    e2 = jnp.min(jnp.where(msk2 == v2, il, 99), axis=1, keepdims=True)

    w1 = jnp.sum(jnp.where(il == e1, scores, 0.0), axis=1, keepdims=True)
    w2 = jnp.sum(jnp.where(il == e2, scores, 0.0), axis=1, keepdims=True)
    denom = w1 + w2 + 1e-20
    w1_ref[...] = ROUTED_SCALING * w1 / denom
    w2_ref[...] = ROUTED_SCALING * w2 / denom

    sel1 = jnp.where(il == e1, 1.0, 0.0)
    sel2 = jnp.where(il == e2, 1.0, 0.0)
    s = sel1 + sel2
    excl = jnp.concatenate([jnp.zeros((1, E), jnp.float32), s[:-1]], axis=0)
    k = 1
    while k < T:
        shifted = jnp.concatenate(
            [jnp.zeros((k, E), jnp.float32), excl[:T - k]], axis=0)
        excl = excl + shifted
        k *= 2
    counts = jnp.sum(s, axis=0, keepdims=True)
    tiles = (counts.astype(jnp.int32) + (BM - 1)) // BM
    tiles_f = tiles.astype(jnp.float32)
    r16 = lax.broadcasted_iota(jnp.int32, (E, E), 0)
    c16 = lax.broadcasted_iota(jnp.int32, (E, E), 1)
    mlt = jnp.where(r16 < c16, 1.0, 0.0)
    base = lax.dot_general(tiles_f, mlt, (((1,), (0,)), ((), ())),
                           preferred_element_type=jnp.float32)
    aligned = base * float(BM)

    rank1 = jnp.sum(jnp.where(il == e1, excl, 0.0), axis=1, keepdims=True)
    off1 = jnp.sum(jnp.where(il == e1, aligned, 0.0), axis=1, keepdims=True)
    p1_ref[...] = (rank1 + off1).astype(jnp.int32)
    rank2 = jnp.sum(jnp.where(il == e2, excl, 0.0), axis=1, keepdims=True)
    off2 = jnp.sum(jnp.where(il == e2, aligned, 0.0), axis=1, keepdims=True)
    p2_ref[...] = (rank2 + off2).astype(jnp.int32)

    used_ref[...] = jnp.sum(tiles, axis=1, keepdims=True)

    it = lax.broadcasted_iota(jnp.int32, (NTILES, 1), 0).astype(jnp.float32)
    cnt = jnp.sum(jnp.where(base <= it, 1, 0), axis=1, keepdims=True)
    te_ref[...] = cnt - 1

    lo_i = lax.bitcast_convert_type(
        x[:, :H // 2].astype(jnp.bfloat16).astype(jnp.float32), jnp.int32)
    hi_i = lax.bitcast_convert_type(
        x[:, H // 2:].astype(jnp.bfloat16).astype(jnp.float32), jnp.int32)
    xi_ref[...] = lax.shift_right_logical(lo_i, 16) | hi_i


def _scatter_body(x_hbm, p1_hbm, p2_hbm, xg_hbm,
                  idx1_v, idx2_v, rows_v, sem1, sem2):
    wid = lax.axis_index("s") * NC + lax.axis_index("c")
    base = wid * TPW
    sl = pl.ds(base, TPW)
    pltpu.sync_copy(p1_hbm.at[sl], idx1_v)
    pltpu.sync_copy(p2_hbm.at[sl], idx2_v)
    pltpu.sync_copy(x_hbm.at[sl], rows_v)
    c1 = pltpu.async_copy(rows_v, xg_hbm.at[idx1_v], sem1)
    c2 = pltpu.async_copy(rows_v, xg_hbm.at[idx2_v], sem2)
    c1.wait()
    c2.wait()


def _gmm_body(te_ref, used_ref, xg_ref, wgu_ref, wd_ref, yg_ref):
    i = pl.program_id(0)

    @pl.when(i < used_ref[0])
    def _():
        xi = xg_ref[...]
        xlo = lax.bitcast_convert_type(
            lax.shift_left(xi, 16), jnp.float32).astype(jnp.bfloat16)
        xhi = lax.bitcast_convert_type(
            xi & jnp.int32(-65536), jnp.float32).astype(jnp.bfloat16)
        wgu = wgu_ref[0]
        gu = (lax.dot_general(xlo, wgu[:, :H // 2],
                              (((1,), (1,)), ((), ())),
                              preferred_element_type=jnp.float32)
              + lax.dot_general(xhi, wgu[:, H // 2:],
                                (((1,), (1,)), ((), ())),
                                preferred_element_type=jnp.float32))
        g = gu[:, :I]
        u = gu[:, I:]
        h = (g * jax.nn.sigmoid(g) * u).astype(jnp.bfloat16)
        wd = wd_ref[0]
        eo = lax.dot_general(h, wd, (((1,), (1,)), ((), ())),
                             preferred_element_type=jnp.float32)
        yg_ref[...] = eo


def _shared_body(x_ref, sgu_ref, sd_ref, out_ref):
    x = x_ref[...]
    sgu = sgu_ref[...]
    gu = lax.dot_general(x, sgu, (((1,), (1,)), ((), ())),
                         preferred_element_type=jnp.float32)
    g = gu[:, :IS]
    u = gu[:, IS:]
    h = (g * jax.nn.sigmoid(g) * u).astype(jnp.bfloat16)
    sd = sd_ref[...]
    out_ref[...] = lax.dot_general(h, sd, (((1,), (1,)), ((), ())),
                                   preferred_element_type=jnp.float32)


def _combine_body(yg_hbm, p1_hbm, p2_hbm, w1_hbm, w2_hbm, sh_hbm, out_hbm,
                  idx1_v, idx2_v, wv1_v, wv2_v, y1_v, y2_v, sh_v, ob_v,
                  sem, sem2, sem3):
    wid = lax.axis_index("s") * NC + lax.axis_index("c")
    base = wid * TPW
    for c in range(TPW // CH):
        b = base + c * CH
        pltpu.sync_copy(p1_hbm.at[pl.ds(b, CH)], idx1_v)
        pltpu.sync_copy(p2_hbm.at[pl.ds(b, CH)], idx2_v)
        pltpu.sync_copy(w1_hbm.at[pl.ds(b, CH)], wv1_v)
        pltpu.sync_copy(w2_hbm.at[pl.ds(b, CH)], wv2_v)
        c1 = pltpu.async_copy(yg_hbm.at[idx1_v], y1_v, sem)
        c2 = pltpu.async_copy(yg_hbm.at[idx2_v], y2_v, sem2)
        c3 = pltpu.async_copy(sh_hbm.at[pl.ds(b, CH)], sh_v, sem3)
        c1.wait()
        c2.wait()
        c3.wait()

        wa = wv1_v[pl.ds(0, CH)]
        wb = wv2_v[pl.ds(0, CH)]

        def body_v(v, carry):
            sl = pl.ds(v * 16, 16)
            for j in range(CH):
                ob_v[j, sl] = (y1_v[j, sl] * wa[j] + y2_v[j, sl] * wb[j]
                               + sh_v[j, sl])
            return carry
        lax.fori_loop(0, H // 16, body_v, 0)
        pltpu.sync_copy(ob_v, out_hbm.at[pl.ds(b, CH)])


@jax.jit
def kernel(hidden_states, gate_w, gate_b, w_gate_up, w_down,
           shared_gate_up, shared_down):
    x = hidden_states
    xb = x.astype(jnp.bfloat16)
    wgu = w_gate_up.astype(jnp.bfloat16)
    wd = w_down.astype(jnp.bfloat16)
    sgu = shared_gate_up.astype(jnp.bfloat16)
    sd = shared_down.astype(jnp.bfloat16)

    p1, p2, w1, w2, te, used, xi = pl.pallas_call(
        _routing_body,
        out_shape=(
            jax.ShapeDtypeStruct((T, 1), jnp.int32),
            jax.ShapeDtypeStruct((T, 1), jnp.int32),
            jax.ShapeDtypeStruct((T, 1), jnp.float32),
            jax.ShapeDtypeStruct((T, 1), jnp.float32),
            jax.ShapeDtypeStruct((NTILES, 1), jnp.int32),
            jax.ShapeDtypeStruct((1, 1), jnp.int32),
            jax.ShapeDtypeStruct((T, H // 2), jnp.int32),
        ),
    )(x, gate_w, gate_b.reshape(1, E))

    p1f = p1.reshape(T)
    p2f = p2.reshape(T)

    shared_out = pl.pallas_call(
        _shared_body,
        grid=(T // 256,),
        in_specs=[
            pl.BlockSpec((256, H), lambda t: (t, 0)),
            pl.BlockSpec((2 * IS, H), lambda t: (0, 0)),
            pl.BlockSpec((H, IS), lambda t: (0, 0)),
        ],
        out_specs=pl.BlockSpec((256, H), lambda t: (t, 0)),
        out_shape=jax.ShapeDtypeStruct((T, H), jnp.float32),
        compiler_params=pltpu.CompilerParams(
            dimension_semantics=("arbitrary",)),
    )(xb, sgu, sd)

    mesh = plsc.VectorSubcoreMesh(core_axis_name="c", subcore_axis_name="s")
    xg = pl.kernel(
        _scatter_body,
        mesh=mesh,
        out_type=jax.ShapeDtypeStruct((PAD_T, H // 2), jnp.int32),
        scratch_types=[
            pltpu.VMEM((TPW,), jnp.int32),
            pltpu.VMEM((TPW,), jnp.int32),
            pltpu.VMEM((TPW, H // 2), jnp.int32),
            pltpu.SemaphoreType.DMA,
            pltpu.SemaphoreType.DMA,
        ],
    )(xi, p1f, p2f)

    yg = pl.pallas_call(
        _gmm_body,
        grid_spec=pltpu.PrefetchScalarGridSpec(
            num_scalar_prefetch=2,
            grid=(NTILES,),
            in_specs=[
                pl.BlockSpec((BM, H // 2), lambda i, te, u: (i, 0)),
                pl.BlockSpec((1, 2 * I, H), lambda i, te, u: (te[i], 0, 0)),
                pl.BlockSpec((1, H, I), lambda i, te, u: (te[i], 0, 0)),
            ],
            out_specs=pl.BlockSpec((BM, H), lambda i, te, u: (i, 0)),
        ),
        out_shape=jax.ShapeDtypeStruct((PAD_T, H), jnp.float32),
        compiler_params=pltpu.CompilerParams(
            dimension_semantics=("arbitrary",)),
    )(te.reshape(NTILES), used.reshape(1), xg, wgu, wd)

    out = pl.kernel(
        _combine_body,
        mesh=mesh,
        out_type=jax.ShapeDtypeStruct((T, H), jnp.float32),
        scratch_types=[
            pltpu.VMEM((CH,), jnp.int32),
            pltpu.VMEM((CH,), jnp.int32),
            pltpu.VMEM((CH,), jnp.float32),
            pltpu.VMEM((CH,), jnp.float32),
            pltpu.VMEM((CH, H), jnp.float32),
            pltpu.VMEM((CH, H), jnp.float32),
            pltpu.VMEM((CH, H), jnp.float32),
            pltpu.VMEM((CH, H), jnp.float32),
            pltpu.SemaphoreType.DMA,
            pltpu.SemaphoreType.DMA,
            pltpu.SemaphoreType.DMA,
        ],
    )(yg, p1f, p2f, w1.reshape(T), w2.reshape(T), shared_out)

    return out


# speedup vs baseline: 1.4654x; 1.0498x over previous
import functools

import jax
import jax.numpy as jnp
from jax import lax
from jax.experimental import pallas as pl
from jax.experimental.pallas import tpu as pltpu
from jax.experimental.pallas import tpu_sc as plsc

T = 2048
H = 1024
E = 16
N_GROUP = 4
GSZ = E // N_GROUP
I = 512
IS = 1024
ROUTED_SCALING = 2.5

BM = 256
NTILES = 32
PAD_T = NTILES * BM

NC = 2
NS = 16
NW = NC * NS
TPW = T // NW
CH = 16

_NEG = -1e30


def _routing_body(x_ref, gw_ref, gb_ref,
                  p1_ref, p2_ref, w1_ref, w2_ref, te_ref, used_ref, xi_ref):
    x = x_ref[...]
    gw = gw_ref[...]
    gb = gb_ref[...]
    logits = lax.dot_general(x, gw, (((1,), (1,)), ((), ())),
                             preferred_element_type=jnp.float32)
    logits = logits + gb
    scores = jax.nn.sigmoid(logits)
    sfc = scores + gb

    il = lax.broadcasted_iota(jnp.int32, (T, E), 1)
    gid = il // GSZ

    gs_cols = []
    for g in range(N_GROUP):
        mg = gid == g
        m1 = jnp.max(jnp.where(mg, sfc, _NEG), axis=1, keepdims=True)
        idx1 = jnp.min(jnp.where(mg & (sfc == m1), il, 99), axis=1,
                       keepdims=True)
        m2 = jnp.max(jnp.where(mg & (il != idx1), sfc, _NEG), axis=1,
                     keepdims=True)
        gs_cols.append(m1 + m2)
    gs = jnp.concatenate(gs_cols, axis=1)

    i4 = lax.broadcasted_iota(jnp.int32, (T, N_GROUP), 1)
    gm1 = jnp.max(gs, axis=1, keepdims=True)
    gi1 = jnp.min(jnp.where(gs == gm1, i4, 99), axis=1, keepdims=True)
    gs2 = jnp.where(i4 == gi1, _NEG, gs)
    gm2 = jnp.max(gs2, axis=1, keepdims=True)
    gi2 = jnp.min(jnp.where(gs2 == gm2, i4, 99), axis=1, keepdims=True)

    allowed = (gid == gi1) | (gid == gi2)
    msk = jnp.where(allowed, sfc, _NEG)

    v1 = jnp.max(msk, axis=1, keepdims=True)
    e1 = jnp.min(jnp.where(msk == v1, il, 99), axis=1, keepdims=True)
    msk2 = jnp.where(il == e1, _NEG, msk)
    v2 = jnp.max(msk2, axis=1, keepdims=True)
    e2 = jnp.min(jnp.where(msk2 == v2, il, 99), axis=1, keepdims=True)

    w1 = jnp.sum(jnp.where(il == e1, scores, 0.0), axis=1, keepdims=True)
    w2 = jnp.sum(jnp.where(il == e2, scores, 0.0), axis=1, keepdims=True)
    denom = w1 + w2 + 1e-20
    w1_ref[...] = ROUTED_SCALING * w1 / denom
    w2_ref[...] = ROUTED_SCALING * w2 / denom

    sel1 = jnp.where(il == e1, 1.0, 0.0)
    sel2 = jnp.where(il == e2, 1.0, 0.0)
    s = sel1 + sel2
    excl = jnp.concatenate([jnp.zeros((1, E), jnp.float32), s[:-1]], axis=0)
    k = 1
    while k < T:
        shifted = jnp.concatenate(
            [jnp.zeros((k, E), jnp.float32), excl[:T - k]], axis=0)
        excl = excl + shifted
        k *= 2
    counts = jnp.sum(s, axis=0, keepdims=True)
    tiles = (counts.astype(jnp.int32) + (BM - 1)) // BM
    tiles_f = tiles.astype(jnp.float32)
    r16 = lax.broadcasted_iota(jnp.int32, (E, E), 0)
    c16 = lax.broadcasted_iota(jnp.int32, (E, E), 1)
    mlt = jnp.where(r16 < c16, 1.0, 0.0)
    base = lax.dot_general(tiles_f, mlt, (((1,), (0,)), ((), ())),
                           preferred_element_type=jnp.float32)
    aligned = base * float(BM)

    rank1 = jnp.sum(jnp.where(il == e1, excl, 0.0), axis=1, keepdims=True)
    off1 = jnp.sum(jnp.where(il == e1, aligned, 0.0), axis=1, keepdims=True)
    p1_ref[...] = (rank1 + off1).astype(jnp.int32)
    rank2 = jnp.sum(jnp.where(il == e2, excl, 0.0), axis=1, keepdims=True)
    off2 = jnp.sum(jnp.where(il == e2, aligned, 0.0), axis=1, keepdims=True)
    p2_ref[...] = (rank2 + off2).astype(jnp.int32)

    used_ref[...] = jnp.sum(tiles, axis=1, keepdims=True)

    it = lax.broadcasted_iota(jnp.int32, (NTILES, 1), 0).astype(jnp.float32)
    cnt = jnp.sum(jnp.where(base <= it, 1, 0), axis=1, keepdims=True)
    te_ref[...] = cnt - 1

    lo_i = lax.bitcast_convert_type(
        x[:, :H // 2].astype(jnp.bfloat16).astype(jnp.float32), jnp.int32)
    hi_i = lax.bitcast_convert_type(
        x[:, H // 2:].astype(jnp.bfloat16).astype(jnp.float32), jnp.int32)
    xi_ref[...] = lax.shift_right_logical(lo_i, 16) | hi_i


def _scatter_body(x_hbm, p1_hbm, p2_hbm, xg_hbm,
                  idx1_v, idx2_v, rows_v, sem1, sem2):
    wid = lax.axis_index("s") * NC + lax.axis_index("c")
    base = wid * TPW
    sl = pl.ds(base, TPW)
    pltpu.sync_copy(p1_hbm.at[sl], idx1_v)
    pltpu.sync_copy(p2_hbm.at[sl], idx2_v)
    pltpu.sync_copy(x_hbm.at[sl], rows_v)
    c1 = pltpu.async_copy(rows_v, xg_hbm.at[idx1_v], sem1)
    c2 = pltpu.async_copy(rows_v, xg_hbm.at[idx2_v], sem2)
    c1.wait()
    c2.wait()


def _gmm_body(te_ref, used_ref, xg_ref, wgu_ref, wd_ref, yg_ref):
    i = pl.program_id(0)

    @pl.when(i < used_ref[0])
    def _():
        xi = xg_ref[...]
        xlo = lax.bitcast_convert_type(
            lax.shift_left(xi, 16), jnp.float32).astype(jnp.bfloat16)
        xhi = lax.bitcast_convert_type(
            xi & jnp.int32(-65536), jnp.float32).astype(jnp.bfloat16)
        wgu = wgu_ref[0]
        gu = (lax.dot_general(xlo, wgu[:, :H // 2],
                              (((1,), (1,)), ((), ())),
                              preferred_element_type=jnp.float32)
              + lax.dot_general(xhi, wgu[:, H // 2:],
                                (((1,), (1,)), ((), ())),
                                preferred_element_type=jnp.float32))
        g = gu[:, :I]
        u = gu[:, I:]
        h = (g * jax.nn.sigmoid(g) * u).astype(jnp.bfloat16)
        wd = wd_ref[0]
        eo = lax.dot_general(h, wd, (((1,), (1,)), ((), ())),
                             preferred_element_type=jnp.float32)
        yg_ref[...] = eo


def _shared_body(x_ref, sgu_ref, sd_ref, out_ref):
    x = x_ref[...]
    sgu = sgu_ref[...]
    gu = lax.dot_general(x, sgu, (((1,), (1,)), ((), ())),
                         preferred_element_type=jnp.float32)
    g = gu[:, :IS]
    u = gu[:, IS:]
    h = (g * jax.nn.sigmoid(g) * u).astype(jnp.bfloat16)
    sd = sd_ref[...]
    out_ref[...] = lax.dot_general(h, sd, (((1,), (1,)), ((), ())),
                                   preferred_element_type=jnp.float32)


def _combine_body(yg_hbm, p1_hbm, p2_hbm, w1_hbm, w2_hbm, sh_hbm, out_hbm,
                  idx1a, idx2a, wv1a, wv2a, y1a, y2a, sha,
                  idx1b, idx2b, wv1b, wv2b, y1b, y2b, shb,
                  ob_v, sem_a, sem_b):
    wid = lax.axis_index("s") * NC + lax.axis_index("c")
    base = wid * TPW
    sets = [(idx1a, idx2a, wv1a, wv2a, y1a, y2a, sha, sem_a),
            (idx1b, idx2b, wv1b, wv2b, y1b, y2b, shb, sem_b)]

    def issue(c, st):
        i1, i2, wv1, wv2, y1, y2, sh, sem = st
        b = base + c * CH
        pltpu.sync_copy(p1_hbm.at[pl.ds(b, CH)], i1)
        pltpu.sync_copy(p2_hbm.at[pl.ds(b, CH)], i2)
        pltpu.sync_copy(w1_hbm.at[pl.ds(b, CH)], wv1)
        pltpu.sync_copy(w2_hbm.at[pl.ds(b, CH)], wv2)
        return (pltpu.async_copy(yg_hbm.at[i1], y1, sem),
                pltpu.async_copy(yg_hbm.at[i2], y2, sem),
                pltpu.async_copy(sh_hbm.at[pl.ds(b, CH)], sh, sem))

    nch = TPW // CH
    pend = issue(0, sets[0])
    for c in range(nch):
        nxt = issue(c + 1, sets[(c + 1) % 2]) if c + 1 < nch else None
        for d in pend:
            d.wait()
        _, _, wv1, wv2, y1, y2, sh, _ = sets[c % 2]
        wa = wv1[pl.ds(0, CH)]
        wb = wv2[pl.ds(0, CH)]

        def body_v(v, carry):
            sl = pl.ds(v * 16, 16)
            for j in range(CH):
                ob_v[j, sl] = (y1[j, sl] * wa[j] + y2[j, sl] * wb[j]
                               + sh[j, sl])
            return carry
        lax.fori_loop(0, H // 16, body_v, 0)
        pltpu.sync_copy(ob_v, out_hbm.at[pl.ds(base + c * CH, CH)])
        pend = nxt


@jax.jit
def kernel(hidden_states, gate_w, gate_b, w_gate_up, w_down,
           shared_gate_up, shared_down):
    x = hidden_states
    xb = x.astype(jnp.bfloat16)
    wgu = w_gate_up.astype(jnp.bfloat16)
    wd = w_down.astype(jnp.bfloat16)
    sgu = shared_gate_up.astype(jnp.bfloat16)
    sd = shared_down.astype(jnp.bfloat16)

    p1, p2, w1, w2, te, used, xi = pl.pallas_call(
        _routing_body,
        out_shape=(
            jax.ShapeDtypeStruct((T, 1), jnp.int32),
            jax.ShapeDtypeStruct((T, 1), jnp.int32),
            jax.ShapeDtypeStruct((T, 1), jnp.float32),
            jax.ShapeDtypeStruct((T, 1), jnp.float32),
            jax.ShapeDtypeStruct((NTILES, 1), jnp.int32),
            jax.ShapeDtypeStruct((1, 1), jnp.int32),
            jax.ShapeDtypeStruct((T, H // 2), jnp.int32),
        ),
    )(x, gate_w, gate_b.reshape(1, E))

    p1f = p1.reshape(T)
    p2f = p2.reshape(T)

    shared_out = pl.pallas_call(
        _shared_body,
        grid=(T // 256,),
        in_specs=[
            pl.BlockSpec((256, H), lambda t: (t, 0)),
            pl.BlockSpec((2 * IS, H), lambda t: (0, 0)),
            pl.BlockSpec((H, IS), lambda t: (0, 0)),
        ],
        out_specs=pl.BlockSpec((256, H), lambda t: (t, 0)),
        out_shape=jax.ShapeDtypeStruct((T, H), jnp.float32),
        compiler_params=pltpu.CompilerParams(
            dimension_semantics=("arbitrary",)),
    )(xb, sgu, sd)

    mesh = plsc.VectorSubcoreMesh(core_axis_name="c", subcore_axis_name="s")
    xg = pl.kernel(
        _scatter_body,
        mesh=mesh,
        out_type=jax.ShapeDtypeStruct((PAD_T, H // 2), jnp.int32),
        scratch_types=[
            pltpu.VMEM((TPW,), jnp.int32),
            pltpu.VMEM((TPW,), jnp.int32),
            pltpu.VMEM((TPW, H // 2), jnp.int32),
            pltpu.SemaphoreType.DMA,
            pltpu.SemaphoreType.DMA,
        ],
    )(xi, p1f, p2f)

    yg = pl.pallas_call(
        _gmm_body,
        grid_spec=pltpu.PrefetchScalarGridSpec(
            num_scalar_prefetch=2,
            grid=(NTILES,),
            in_specs=[
                pl.BlockSpec((BM, H // 2), lambda i, te, u: (i, 0)),
                pl.BlockSpec((1, 2 * I, H), lambda i, te, u: (te[i], 0, 0)),
                pl.BlockSpec((1, H, I), lambda i, te, u: (te[i], 0, 0)),
            ],
            out_specs=pl.BlockSpec((BM, H), lambda i, te, u: (i, 0)),
        ),
        out_shape=jax.ShapeDtypeStruct((PAD_T, H), jnp.float32),
        compiler_params=pltpu.CompilerParams(
            dimension_semantics=("arbitrary",)),
    )(te.reshape(NTILES), used.reshape(1), xg, wgu, wd)

    out = pl.kernel(
        _combine_body,
        mesh=mesh,
        out_type=jax.ShapeDtypeStruct((T, H), jnp.float32),
        scratch_types=(
            [pltpu.VMEM((CH,), jnp.int32)] * 2
            + [pltpu.VMEM((CH,), jnp.float32)] * 2
            + [pltpu.VMEM((CH, H), jnp.float32)] * 3
            + [pltpu.VMEM((CH,), jnp.int32)] * 2
            + [pltpu.VMEM((CH,), jnp.float32)] * 2
            + [pltpu.VMEM((CH, H), jnp.float32)] * 3
            + [pltpu.VMEM((CH, H), jnp.float32)]
            + [pltpu.SemaphoreType.DMA] * 2
        ),
    )(yg, p1f, p2f, w1.reshape(T), w2.reshape(T), shared_out)

    return out


# speedup vs baseline: 1.4667x; 1.0009x over previous
import jax
import jax.numpy as jnp
from jax import lax
from jax.experimental import pallas as pl
from jax.experimental.pallas import tpu as pltpu
from jax.experimental.pallas import tpu_sc as plsc

T = 2048
H = 1024
E = 16
N_GROUP = 4
GSZ = E // N_GROUP
I = 512
IS = 1024
ROUTED_SCALING = 2.5

BM = 256
NTILES = 32
PAD_T = NTILES * BM

NC = 2
NS = 16
NW = NC * NS
TPW = T // NW
CH = 16

_NEG = -1e30


def _routing_body(x_ref, gw_ref, gb_ref,
                  p1_ref, p2_ref, w1_ref, w2_ref, te_ref, used_ref, xi_ref):
    x = x_ref[...]
    gw = gw_ref[...]
    gb = gb_ref[...]
    logits = lax.dot_general(x, gw, (((1,), (1,)), ((), ())),
                             preferred_element_type=jnp.float32)
    logits = logits + gb
    scores = jax.nn.sigmoid(logits)
    sfc = scores + gb

    il = lax.broadcasted_iota(jnp.int32, (T, E), 1)
    gid = il // GSZ

    gs_cols = []
    for g in range(N_GROUP):
        mg = gid == g
        m1 = jnp.max(jnp.where(mg, sfc, _NEG), axis=1, keepdims=True)
        idx1 = jnp.min(jnp.where(mg & (sfc == m1), il, 99), axis=1,
                       keepdims=True)
        m2 = jnp.max(jnp.where(mg & (il != idx1), sfc, _NEG), axis=1,
                     keepdims=True)
        gs_cols.append(m1 + m2)
    gs = jnp.concatenate(gs_cols, axis=1)

    i4 = lax.broadcasted_iota(jnp.int32, (T, N_GROUP), 1)
    gm1 = jnp.max(gs, axis=1, keepdims=True)
    gi1 = jnp.min(jnp.where(gs == gm1, i4, 99), axis=1, keepdims=True)
    gs2 = jnp.where(i4 == gi1, _NEG, gs)
    gm2 = jnp.max(gs2, axis=1, keepdims=True)
    gi2 = jnp.min(jnp.where(gs2 == gm2, i4, 99), axis=1, keepdims=True)

    allowed = (gid == gi1) | (gid == gi2)
    msk = jnp.where(allowed, sfc, _NEG)

    v1 = jnp.max(msk, axis=1, keepdims=True)
    e1 = jnp.min(jnp.where(msk == v1, il, 99), axis=1, keepdims=True)
    msk2 = jnp.where(il == e1, _NEG, msk)
    v2 = jnp.max(msk2, axis=1, keepdims=True)
    e2 = jnp.min(jnp.where(msk2 == v2, il, 99), axis=1, keepdims=True)

    w1 = jnp.sum(jnp.where(il == e1, scores, 0.0), axis=1, keepdims=True)
    w2 = jnp.sum(jnp.where(il == e2, scores, 0.0), axis=1, keepdims=True)
    denom = w1 + w2 + 1e-20
    w1_ref[...] = ROUTED_SCALING * w1 / denom
    w2_ref[...] = ROUTED_SCALING * w2 / denom

    sel1 = jnp.where(il == e1, 1.0, 0.0)
    sel2 = jnp.where(il == e2, 1.0, 0.0)
    s = sel1 + sel2
    excl = jnp.concatenate([jnp.zeros((1, E), jnp.float32), s[:-1]], axis=0)
    k = 1
    while k < T:
        shifted = jnp.concatenate(
            [jnp.zeros((k, E), jnp.float32), excl[:T - k]], axis=0)
        excl = excl + shifted
        k *= 2
    counts = jnp.sum(s, axis=0, keepdims=True)
    tiles = (counts.astype(jnp.int32) + (BM - 1)) // BM
    tiles_f = tiles.astype(jnp.float32)
    r16 = lax.broadcasted_iota(jnp.int32, (E, E), 0)
    c16 = lax.broadcasted_iota(jnp.int32, (E, E), 1)
    mlt = jnp.where(r16 < c16, 1.0, 0.0)
    base = lax.dot_general(tiles_f, mlt, (((1,), (0,)), ((), ())),
                           preferred_element_type=jnp.float32)
    aligned = base * float(BM)

    rank1 = jnp.sum(jnp.where(il == e1, excl, 0.0), axis=1, keepdims=True)
    off1 = jnp.sum(jnp.where(il == e1, aligned, 0.0), axis=1, keepdims=True)
    p1_ref[...] = (rank1 + off1).astype(jnp.int32)
    rank2 = jnp.sum(jnp.where(il == e2, excl, 0.0), axis=1, keepdims=True)
    off2 = jnp.sum(jnp.where(il == e2, aligned, 0.0), axis=1, keepdims=True)
    p2_ref[...] = (rank2 + off2).astype(jnp.int32)

    used_ref[...] = jnp.sum(tiles, axis=1, keepdims=True)

    it = lax.broadcasted_iota(jnp.int32, (NTILES, 1), 0).astype(jnp.float32)
    cnt = jnp.sum(jnp.where(base <= it, 1, 0), axis=1, keepdims=True)
    te_ref[...] = cnt - 1

    lo_i = lax.bitcast_convert_type(
        x[:, :H // 2].astype(jnp.bfloat16).astype(jnp.float32), jnp.int32)
    hi_i = lax.bitcast_convert_type(
        x[:, H // 2:].astype(jnp.bfloat16).astype(jnp.float32), jnp.int32)
    xi_ref[...] = lax.shift_right_logical(lo_i, 16) | hi_i


def _scatter_body(x_hbm, p1_hbm, p2_hbm, xg_hbm,
                  idx1_v, idx2_v, rows_v, sem1, sem2):
    wid = lax.axis_index("s") * NC + lax.axis_index("c")
    base = wid * TPW
    sl = pl.ds(base, TPW)
    pltpu.sync_copy(p1_hbm.at[sl], idx1_v)
    pltpu.sync_copy(p2_hbm.at[sl], idx2_v)
    pltpu.sync_copy(x_hbm.at[sl], rows_v)
    c1 = pltpu.async_copy(rows_v, xg_hbm.at[idx1_v], sem1)
    c2 = pltpu.async_copy(rows_v, xg_hbm.at[idx2_v], sem2)
    c1.wait()
    c2.wait()


def _gmm_body(te_ref, used_ref, xg_ref, wgu_ref, wd_ref, yg_ref):
    i = pl.program_id(0)

    @pl.when(i < used_ref[0])
    def _():
        xi = xg_ref[...]
        xlo = lax.bitcast_convert_type(
            lax.shift_left(xi, 16), jnp.float32).astype(jnp.bfloat16)
        xhi = lax.bitcast_convert_type(
            xi & jnp.int32(-65536), jnp.float32).astype(jnp.bfloat16)
        wgu = wgu_ref[0]
        gu = (lax.dot_general(xlo, wgu[:, :H // 2],
                              (((1,), (1,)), ((), ())),
                              preferred_element_type=jnp.float32)
              + lax.dot_general(xhi, wgu[:, H // 2:],
                                (((1,), (1,)), ((), ())),
                                preferred_element_type=jnp.float32))
        g = gu[:, :I]
        u = gu[:, I:]
        h = (g * jax.nn.sigmoid(g) * u).astype(jnp.bfloat16)
        wd = wd_ref[0]
        eo = lax.dot_general(h, wd, (((1,), (1,)), ((), ())),
                             preferred_element_type=jnp.float32)
        yg_ref[...] = eo


def _shared_body(x_ref, sgu_ref, sd_ref, out_ref):
    x = x_ref[...]
    sgu = sgu_ref[...]
    gu = lax.dot_general(x, sgu, (((1,), (1,)), ((), ())),
                         preferred_element_type=jnp.float32)
    g = gu[:, :IS]
    u = gu[:, IS:]
    h = (g * jax.nn.sigmoid(g) * u).astype(jnp.bfloat16)
    sd = sd_ref[...]
    out_ref[...] = lax.dot_general(h, sd, (((1,), (1,)), ((), ())),
                                   preferred_element_type=jnp.float32)


def _combine_body(yg_hbm, p1_hbm, p2_hbm, w1_hbm, w2_hbm, sh_hbm, out_hbm,
                  idx1a, idx2a, wv1a, wv2a, y1a, y2a, sha,
                  idx1b, idx2b, wv1b, wv2b, y1b, y2b, shb,
                  ob_v, sem_a, sem_b):
    wid = lax.axis_index("s") * NC + lax.axis_index("c")
    base = wid * TPW
    sets = [(idx1a, idx2a, wv1a, wv2a, y1a, y2a, sha, sem_a),
            (idx1b, idx2b, wv1b, wv2b, y1b, y2b, shb, sem_b)]

    def issue(c, st):
        i1, i2, wv1, wv2, y1, y2, sh, sem = st
        b = base + c * CH
        pltpu.sync_copy(p1_hbm.at[pl.ds(b, CH)], i1)
        pltpu.sync_copy(p2_hbm.at[pl.ds(b, CH)], i2)
        pltpu.sync_copy(w1_hbm.at[pl.ds(b, CH)], wv1)
        pltpu.sync_copy(w2_hbm.at[pl.ds(b, CH)], wv2)
        return (pltpu.async_copy(yg_hbm.at[i1], y1, sem),
                pltpu.async_copy(yg_hbm.at[i2], y2, sem),
                pltpu.async_copy(sh_hbm.at[pl.ds(b, CH)], sh, sem))

    nch = TPW // CH
    pend = issue(0, sets[0])
    for c in range(nch):
        nxt = issue(c + 1, sets[(c + 1) % 2]) if c + 1 < nch else None
        for d in pend:
            d.wait()
        _, _, wv1, wv2, y1, y2, sh, _ = sets[c % 2]
        wa = wv1[pl.ds(0, CH)]
        wb = wv2[pl.ds(0, CH)]

        def body_v(v, carry):
            sl = pl.ds(v * 16, 16)
            for j in range(CH):
                ob_v[j, sl] = (y1[j, sl] * wa[j] + y2[j, sl] * wb[j]
                               + sh[j, sl])
            return carry
        lax.fori_loop(0, H // 16, body_v, 0)
        pltpu.sync_copy(ob_v, out_hbm.at[pl.ds(base + c * CH, CH)])
        pend = nxt


@jax.jit
def kernel(hidden_states, gate_w, gate_b, w_gate_up, w_down,
           shared_gate_up, shared_down):
    x = hidden_states
    xb = x.astype(jnp.bfloat16)
    wgu = w_gate_up.astype(jnp.bfloat16)
    wd = w_down.astype(jnp.bfloat16)
    sgu = shared_gate_up.astype(jnp.bfloat16)
    sd = shared_down.astype(jnp.bfloat16)

    p1, p2, w1, w2, te, used, xi = pl.pallas_call(
        _routing_body,
        out_shape=(
            jax.ShapeDtypeStruct((T, 1), jnp.int32),
            jax.ShapeDtypeStruct((T, 1), jnp.int32),
            jax.ShapeDtypeStruct((T, 1), jnp.float32),
            jax.ShapeDtypeStruct((T, 1), jnp.float32),
            jax.ShapeDtypeStruct((NTILES, 1), jnp.int32),
            jax.ShapeDtypeStruct((1, 1), jnp.int32),
            jax.ShapeDtypeStruct((T, H // 2), jnp.int32),
        ),
    )(x, gate_w, gate_b.reshape(1, E))

    p1f = p1.reshape(T)
    p2f = p2.reshape(T)

    shared_out = pl.pallas_call(
        _shared_body,
        grid=(T // 256,),
        in_specs=[
            pl.BlockSpec((256, H), lambda t: (t, 0)),
            pl.BlockSpec((2 * IS, H), lambda t: (0, 0)),
            pl.BlockSpec((H, IS), lambda t: (0, 0)),
        ],
        out_specs=pl.BlockSpec((256, H), lambda t: (t, 0)),
        out_shape=jax.ShapeDtypeStruct((T, H), jnp.float32),
        compiler_params=pltpu.CompilerParams(
            dimension_semantics=("arbitrary",)),
    )(xb, sgu, sd)

    mesh = plsc.VectorSubcoreMesh(core_axis_name="c", subcore_axis_name="s")
    xg = pl.kernel(
        _scatter_body,
        mesh=mesh,
        out_type=jax.ShapeDtypeStruct((PAD_T, H // 2), jnp.int32),
        scratch_types=[
            pltpu.VMEM((TPW,), jnp.int32),
            pltpu.VMEM((TPW,), jnp.int32),
            pltpu.VMEM((TPW, H // 2), jnp.int32),
            pltpu.SemaphoreType.DMA,
            pltpu.SemaphoreType.DMA,
        ],
    )(xi, p1f, p2f)

    yg = pl.pallas_call(
        _gmm_body,
        grid_spec=pltpu.PrefetchScalarGridSpec(
            num_scalar_prefetch=2,
            grid=(NTILES,),
            in_specs=[
                pl.BlockSpec((BM, H // 2), lambda i, te, u: (i, 0)),
                pl.BlockSpec((1, 2 * I, H), lambda i, te, u: (te[i], 0, 0)),
                pl.BlockSpec((1, H, I), lambda i, te, u: (te[i], 0, 0)),
            ],
            out_specs=pl.BlockSpec((BM, H), lambda i, te, u: (i, 0)),
        ),
        out_shape=jax.ShapeDtypeStruct((PAD_T, H), jnp.float32),
        compiler_params=pltpu.CompilerParams(
            dimension_semantics=("arbitrary",)),
    )(te.reshape(NTILES), used.reshape(1), xg, wgu, wd)

    out = pl.kernel(
        _combine_body,
        mesh=mesh,
        out_type=jax.ShapeDtypeStruct((T, H), jnp.float32),
        scratch_types=(
            [pltpu.VMEM((CH,), jnp.int32)] * 2
            + [pltpu.VMEM((CH,), jnp.float32)] * 2
            + [pltpu.VMEM((CH, H), jnp.float32)] * 3
            + [pltpu.VMEM((CH,), jnp.int32)] * 2
            + [pltpu.VMEM((CH,), jnp.float32)] * 2
            + [pltpu.VMEM((CH, H), jnp.float32)] * 3
            + [pltpu.VMEM((CH, H), jnp.float32)]
            + [pltpu.SemaphoreType.DMA] * 2
        ),
    )(yg, p1f, p2f, w1.reshape(T), w2.reshape(T), shared_out)

    return out
